# baseline (speedup 1.0000x reference)
import functools

import jax
import jax.numpy as jnp
from jax import lax
from jax.experimental import pallas as pl
from jax.experimental.pallas import tpu as pltpu
from jax.experimental.pallas import tpu_sc as plsc

NC = 2
NS = 16
NW = NC * NS
LANES = 16

N = 10000
E = 320000
C = 80
EW = E // NW
G = EW // C
NPAD = 10240
RPS = NPAD // NS
NROW = N // 8
PPAD = NPAD // 8

NB = 12
PD = 6


def _sc_mesh():
    return plsc.VectorSubcoreMesh(
        core_axis_name="c", subcore_axis_name="s",
        num_cores=NC, num_subcores=NS)


def _deg_body(z_hbm, e_hbm, out_hbm, dst_slab, ones_v, acc, sem, zsem):
    c = lax.axis_index("c")
    s = lax.axis_index("s")
    wid = c * NS + s
    zero_copy = pltpu.async_copy(z_hbm, acc.at[pl.ds(s * RPS, RPS)], zsem)
    pltpu.sync_copy(e_hbm.at[1].at[wid], dst_slab)

    def orow(i, _):
        ones_v[i] = jnp.ones((LANES,), jnp.float32)
        return 0
    lax.fori_loop(0, C, orow, 0)

    zero_copy.wait()
    plsc.subcore_barrier()

    def drain():
        pltpu.make_async_copy(out_hbm.at[pl.ds(0, C)], ones_v, sem).wait()

    def chunk_a(g, _):
        pltpu.async_copy(ones_v, acc.at[dst_slab.at[g]], sem, add=True)
        return 0

    def chunk_b(g, _):
        pltpu.async_copy(ones_v, acc.at[dst_slab.at[g]], sem, add=True)
        drain()
        return 0
    lax.fori_loop(0, NB, chunk_a, 0)
    lax.fori_loop(NB, G, chunk_b, 0)
    for _ in range(NB):
        drain()
    plsc.subcore_barrier()

    pltpu.sync_copy(acc.at[pl.ds(s * RPS, RPS)],
                    out_hbm.at[pl.ds(wid * RPS, RPS)])


@functools.lru_cache(maxsize=None)
def _deg_kernel():
    return pl.kernel(
        _deg_body,
        out_type=jax.ShapeDtypeStruct((NC * NPAD, LANES), jnp.float32),
        mesh=_sc_mesh(),
        scratch_types=[
            pltpu.VMEM((G, C), jnp.int32),
            pltpu.VMEM((C, LANES), jnp.float32),
            pltpu.VMEM_SHARED((NPAD, LANES), jnp.float32),
            pltpu.SemaphoreType.DMA,
            pltpu.SemaphoreType.DMA,
        ],
        compiler_params=pltpu.CompilerParams(use_tc_tiling_on_sc=False),
    )


def _deg_call(z, e3):
    return _deg_kernel()(z, e3)


def _agg_body(z_hbm, hs_hbm, e_hbm, out_hbm,
              src_slab, dst_slab, rows, acc, gsem, ssem, zsem, wsem):
    c = lax.axis_index("c")
    s = lax.axis_index("s")
    wid = c * NS + s
    zero_copy = pltpu.async_copy(z_hbm, acc.at[pl.ds(s * RPS, RPS)], wsem)
    src_load = pltpu.async_copy(e_hbm.at[0].at[wid], src_slab, zsem)
    dst_load = pltpu.async_copy(e_hbm.at[1].at[wid], dst_slab, zsem)
    src_load.wait()
    dst_load.wait()

    def fire_gather(g, b):
        pltpu.async_copy(hs_hbm.at[src_slab.at[g]], rows.at[b], gsem.at[b])

    def drain(sem_arr, b):
        pltpu.make_async_copy(hs_hbm.at[pl.ds(0, C)], rows.at[b],
                              sem_arr.at[b]).wait()

    def scat(g, b):
        pltpu.async_copy(rows.at[b], acc.at[dst_slab.at[g]], ssem.at[b],
                         add=True)

    for g in range(PD):
        fire_gather(jnp.int32(g), jnp.int32(g))
    zero_copy.wait()
    plsc.subcore_barrier()

    def body_a(g, _):
        b = lax.rem(g, NB)
        drain(gsem, b)
        scat(g, b)
        gp = g + PD
        fire_gather(gp, lax.rem(gp, NB))
        return 0

    def body_b(g, _):
        b = lax.rem(g, NB)
        drain(gsem, b)
        scat(g, b)
        gp = g + PD
        bp = lax.rem(gp, NB)
        drain(ssem, bp)
        fire_gather(gp, bp)
        return 0

    def body_c(g, _):
        b = lax.rem(g, NB)
        drain(gsem, b)
        scat(g, b)
        return 0
    lax.fori_loop(0, NB - PD, body_a, 0)
    lax.fori_loop(NB - PD, G - PD, body_b, 0)
    lax.fori_loop(G - PD, G, body_c, 0)
    for k in range(NB):
        drain(ssem, jnp.int32((G - NB + k) % NB))
    plsc.subcore_barrier()

    pltpu.sync_copy(acc.at[pl.ds(s * RPS, RPS)],
                    out_hbm.at[pl.ds(wid * RPS, RPS)])


@functools.lru_cache(maxsize=None)
def _agg_kernel():
    return pl.kernel(
        _agg_body,
        out_type=jax.ShapeDtypeStruct((NC * NPAD, LANES), jnp.float32),
        mesh=_sc_mesh(),
        scratch_types=[
            pltpu.VMEM((G, C), jnp.int32),
            pltpu.VMEM((G, C), jnp.int32),
            pltpu.VMEM((NB, C, LANES), jnp.float32),
            pltpu.VMEM_SHARED((NPAD, LANES), jnp.float32),
            pltpu.SemaphoreType.DMA((NB,)),
            pltpu.SemaphoreType.DMA((NB,)),
            pltpu.SemaphoreType.DMA,
            pltpu.SemaphoreType.DMA,
        ],
        compiler_params=pltpu.CompilerParams(use_tc_tiling_on_sc=False),
    )


def _agg_call(z, hs, e3):
    return _agg_kernel()(z, hs, e3)


def _tc1_body(degp_ref, x3_ref, w_ref, dinv_ref, hs_ref):
    deg = degp_ref[0:NROW, :] + degp_ref[PPAD:PPAD + NROW, :] + 1.0
    dinv = lax.rsqrt(deg)
    dinv_ref[...] = dinv
    for a in range(8):
        h = jnp.dot(x3_ref[:, a, :], w_ref[...],
                    preferred_element_type=jnp.float32)
        hs_ref[:, 16 * a:16 * (a + 1)] = h * dinv[:, 16 * a:16 * (a + 1)]


def _tc1(degp, x3, w):
    return pl.pallas_call(
        _tc1_body,
        out_shape=[
            jax.ShapeDtypeStruct((NROW, 128), jnp.float32),
            jax.ShapeDtypeStruct((NROW, 128), jnp.float32),
        ],
    )(degp, x3, w)


def _rep8(v):
    return jnp.concatenate([v] * 8, axis=0)


def _ln_relu(t, mavg_ref, g_ref, be_ref):
    hi = lax.Precision.HIGHEST
    mavg = mavg_ref[...]
    mu = jnp.dot(t, mavg, precision=hi, preferred_element_type=jnp.float32)
    d = t - mu
    var = jnp.dot(d * d, mavg, precision=hi,
                  preferred_element_type=jnp.float32)
    tn = d * lax.rsqrt(var + 1e-5) * _rep8(g_ref[...])[None, :] \
        + _rep8(be_ref[...])[None, :]
    return jnp.maximum(tn, 0.0)


def _tc_mid_body(aggp_ref, hs_ref, dinv_ref, b_ref, g_ref, be_ref,
                 wb_ref, mavg_ref, out_ref):
    dinv = dinv_ref[...]
    t = (aggp_ref[0:NROW, :] + aggp_ref[PPAD:PPAD + NROW, :] + hs_ref[...]) \
        * dinv + _rep8(b_ref[...])[None, :]
    r = _ln_relu(t, mavg_ref, g_ref, be_ref)
    out_ref[...] = jnp.dot(r, wb_ref[...],
                           preferred_element_type=jnp.float32) * dinv


def _tc_mid(aggp, hs, dinv, b, g, be, wb, mavg):
    return pl.pallas_call(
        _tc_mid_body,
        out_shape=jax.ShapeDtypeStruct((NROW, 128), jnp.float32),
    )(aggp, hs, dinv, b, g, be, wb, mavg)


def _tc_final_body(aggp_ref, hs_ref, dinv_ref, b_ref, g_ref, be_ref,
                   mavg_ref, f_ref, wl_ref, bl_ref, out_ref):
    hi = lax.Precision.HIGHEST
    t = (aggp_ref[0:NROW, :] + aggp_ref[PPAD:PPAD + NROW, :] + hs_ref[...]) \
        * dinv_ref[...] + _rep8(b_ref[...])[None, :]
    r = _ln_relu(t, mavg_ref, g_ref, be_ref)
    srow = jnp.sum(r, axis=0, keepdims=True)
    pooled = jnp.dot(srow, f_ref[...], precision=hi,
                     preferred_element_type=jnp.float32) * (1.0 / N)
    out_ref[...] = jnp.dot(pooled, wl_ref[...],
                           preferred_element_type=jnp.float32) \
        + bl_ref[...][None, :]


def _tc_final(aggp, hs, dinv, b, g, be, mavg, f, wl, bl):
    return pl.pallas_call(
        _tc_final_body,
        out_shape=jax.ShapeDtypeStruct((1, 25), jnp.float32),
    )(aggp, hs, dinv, b, g, be, mavg, f, wl, bl)


def kernel(x, edge_index, W1, b1, g1, be1, W2, b2, g2, be2, W3, b3, g3, be3,
           Wl, bl):
    e3 = edge_index.reshape(2, NW, G, C)
    x3 = x.reshape(NROW, 8, 128)
    eye8 = jnp.eye(8, dtype=jnp.float32)
    mavg = jnp.kron(eye8, jnp.full((16, 16), 1.0 / 16, jnp.float32))
    fmat = jnp.kron(jnp.ones((8, 1), jnp.float32),
                    jnp.eye(16, dtype=jnp.float32))
    zrows = jnp.zeros((RPS, LANES), jnp.float32)

    degp = _deg_call(zrows, e3).reshape(NC * PPAD, 128)
    dinv, hs = _tc1(degp, x3, W1)

    aggp = _agg_call(zrows, hs.reshape(N, LANES), e3).reshape(NC * PPAD, 128)
    hs = _tc_mid(aggp, hs, dinv, b1, g1, be1, jnp.kron(eye8, W2), mavg)

    aggp = _agg_call(zrows, hs.reshape(N, LANES), e3).reshape(NC * PPAD, 128)
    hs = _tc_mid(aggp, hs, dinv, b2, g2, be2, jnp.kron(eye8, W3), mavg)

    aggp = _agg_call(zrows, hs.reshape(N, LANES), e3).reshape(NC * PPAD, 128)
    out = _tc_final(aggp, hs, dinv, b3, g3, be3, mavg, fmat, Wl, bl)
    return out.reshape(25)


# speedup vs baseline: 1.0184x; 1.0184x over previous
import functools

import jax
import jax.numpy as jnp
from jax import lax
from jax.experimental import pallas as pl
from jax.experimental.pallas import tpu as pltpu
from jax.experimental.pallas import tpu_sc as plsc

NC = 2
NS = 16
NW = NC * NS
LANES = 16

N = 10000
E = 320000
C = 80
EW = E // NW
G = EW // C
NPAD = 10240
RPS = NPAD // NS
NROW = N // 8
PPAD = NPAD // 8

NB = 12
PD = 6


def _sc_mesh():
    return plsc.VectorSubcoreMesh(
        core_axis_name="c", subcore_axis_name="s",
        num_cores=NC, num_subcores=NS)


def _deg_body(e_hbm, out_hbm, dst_slab, ones_v, stage, acc, sem):
    c = lax.axis_index("c")
    s = lax.axis_index("s")
    wid = c * NS + s
    dst_load = pltpu.async_copy(e_hbm.at[1].at[wid], dst_slab, sem)

    def zrow(i, _):
        stage[i] = jnp.zeros((LANES,), jnp.float32)
        return 0
    lax.fori_loop(0, RPS, zrow, 0)

    def orow(i, _):
        ones_v[i] = jnp.ones((LANES,), jnp.float32)
        return 0
    lax.fori_loop(0, C, orow, 0)

    pltpu.sync_copy(stage, acc.at[pl.ds(s * RPS, RPS)])
    dst_load.wait()
    plsc.subcore_barrier()

    def drain():
        pltpu.make_async_copy(out_hbm.at[pl.ds(0, C)], ones_v, sem).wait()

    def chunk_a(g, _):
        pltpu.async_copy(ones_v, acc.at[dst_slab.at[g]], sem, add=True)
        return 0

    def chunk_b(g, _):
        pltpu.async_copy(ones_v, acc.at[dst_slab.at[g]], sem, add=True)
        drain()
        return 0
    lax.fori_loop(0, NB, chunk_a, 0)
    lax.fori_loop(NB, G, chunk_b, 0)
    for _ in range(NB):
        drain()
    plsc.subcore_barrier()

    pltpu.sync_copy(acc.at[pl.ds(s * RPS, RPS)],
                    out_hbm.at[pl.ds(wid * RPS, RPS)])


@functools.lru_cache(maxsize=None)
def _deg_kernel():
    return pl.kernel(
        _deg_body,
        out_type=jax.ShapeDtypeStruct((NC * NPAD, LANES), jnp.float32),
        mesh=_sc_mesh(),
        scratch_types=[
            pltpu.VMEM((G, C), jnp.int32),
            pltpu.VMEM((C, LANES), jnp.float32),
            pltpu.VMEM((RPS, LANES), jnp.float32),
            pltpu.VMEM_SHARED((NPAD, LANES), jnp.float32),
            pltpu.SemaphoreType.DMA,
        ],
        compiler_params=pltpu.CompilerParams(use_tc_tiling_on_sc=False),
    )


def _deg_call(e3):
    return _deg_kernel()(e3)


def _agg_body(hs_hbm, e_hbm, out_hbm,
              src_slab, dst_slab, rows, stage, acc, gsem, ssem, zsem):
    c = lax.axis_index("c")
    s = lax.axis_index("s")
    wid = c * NS + s
    src_load = pltpu.async_copy(e_hbm.at[0].at[wid], src_slab, zsem)
    dst_load = pltpu.async_copy(e_hbm.at[1].at[wid], dst_slab, zsem)

    def zrow(i, _):
        stage[i] = jnp.zeros((LANES,), jnp.float32)
        return 0
    lax.fori_loop(0, RPS, zrow, 0)
    src_load.wait()
    dst_load.wait()

    def fire_gather(g, b):
        pltpu.async_copy(hs_hbm.at[src_slab.at[g]], rows.at[b], gsem.at[b])

    def drain(sem_arr, b):
        pltpu.make_async_copy(hs_hbm.at[pl.ds(0, C)], rows.at[b],
                              sem_arr.at[b]).wait()

    def scat(g, b):
        pltpu.async_copy(rows.at[b], acc.at[dst_slab.at[g]], ssem.at[b],
                         add=True)

    zero_copy = pltpu.async_copy(stage, acc.at[pl.ds(s * RPS, RPS)], zsem)
    for g in range(PD):
        fire_gather(jnp.int32(g), jnp.int32(g))
    zero_copy.wait()
    plsc.subcore_barrier()

    def body_a(g, _):
        b = lax.rem(g, NB)
        drain(gsem, b)
        scat(g, b)
        gp = g + PD
        fire_gather(gp, lax.rem(gp, NB))
        return 0

    def body_b(g, _):
        b = lax.rem(g, NB)
        drain(gsem, b)
        scat(g, b)
        gp = g + PD
        bp = lax.rem(gp, NB)
        drain(ssem, bp)
        fire_gather(gp, bp)
        return 0

    def body_c(g, _):
        b = lax.rem(g, NB)
        drain(gsem, b)
        scat(g, b)
        return 0
    lax.fori_loop(0, NB - PD, body_a, 0)
    lax.fori_loop(NB - PD, G - PD, body_b, 0)
    lax.fori_loop(G - PD, G, body_c, 0)
    for k in range(NB):
        drain(ssem, jnp.int32((G - NB + k) % NB))
    plsc.subcore_barrier()

    pltpu.sync_copy(acc.at[pl.ds(s * RPS, RPS)],
                    out_hbm.at[pl.ds(wid * RPS, RPS)])


@functools.lru_cache(maxsize=None)
def _agg_kernel():
    return pl.kernel(
        _agg_body,
        out_type=jax.ShapeDtypeStruct((NC * NPAD, LANES), jnp.float32),
        mesh=_sc_mesh(),
        scratch_types=[
            pltpu.VMEM((G, C), jnp.int32),
            pltpu.VMEM((G, C), jnp.int32),
            pltpu.VMEM((NB, C, LANES), jnp.float32),
            pltpu.VMEM((RPS, LANES), jnp.float32),
            pltpu.VMEM_SHARED((NPAD, LANES), jnp.float32),
            pltpu.SemaphoreType.DMA((NB,)),
            pltpu.SemaphoreType.DMA((NB,)),
            pltpu.SemaphoreType.DMA,
        ],
        compiler_params=pltpu.CompilerParams(use_tc_tiling_on_sc=False),
    )


def _agg_call(hs, e3):
    return _agg_kernel()(hs, e3)


def _tc1_body(degp_ref, x3_ref, w_ref, dinv_ref, hs_ref):
    deg = degp_ref[0:NROW, :] + degp_ref[PPAD:PPAD + NROW, :] + 1.0
    dinv = lax.rsqrt(deg)
    dinv_ref[...] = dinv
    for a in range(8):
        h = jnp.dot(x3_ref[:, a, :], w_ref[...],
                    preferred_element_type=jnp.float32)
        hs_ref[:, 16 * a:16 * (a + 1)] = h * dinv[:, 16 * a:16 * (a + 1)]


def _tc1(degp, x3, w):
    return pl.pallas_call(
        _tc1_body,
        out_shape=[
            jax.ShapeDtypeStruct((NROW, 128), jnp.float32),
            jax.ShapeDtypeStruct((NROW, 128), jnp.float32),
        ],
    )(degp, x3, w)


def _rep8(v):
    return jnp.concatenate([v] * 8, axis=0)


def _ln_relu(t, mavg_ref, g_ref, be_ref):
    hi = lax.Precision.HIGHEST
    mavg = mavg_ref[...]
    mu = jnp.dot(t, mavg, precision=hi, preferred_element_type=jnp.float32)
    d = t - mu
    var = jnp.dot(d * d, mavg, precision=hi,
                  preferred_element_type=jnp.float32)
    tn = d * lax.rsqrt(var + 1e-5) * _rep8(g_ref[...])[None, :] \
        + _rep8(be_ref[...])[None, :]
    return jnp.maximum(tn, 0.0)


def _tc_mid_body(aggp_ref, hs_ref, dinv_ref, b_ref, g_ref, be_ref,
                 wb_ref, mavg_ref, out_ref):
    dinv = dinv_ref[...]
    t = (aggp_ref[0:NROW, :] + aggp_ref[PPAD:PPAD + NROW, :] + hs_ref[...]) \
        * dinv + _rep8(b_ref[...])[None, :]
    r = _ln_relu(t, mavg_ref, g_ref, be_ref)
    out_ref[...] = jnp.dot(r, wb_ref[...],
                           preferred_element_type=jnp.float32) * dinv


def _tc_mid(aggp, hs, dinv, b, g, be, wb, mavg):
    return pl.pallas_call(
        _tc_mid_body,
        out_shape=jax.ShapeDtypeStruct((NROW, 128), jnp.float32),
    )(aggp, hs, dinv, b, g, be, wb, mavg)


def _tc_final_body(aggp_ref, hs_ref, dinv_ref, b_ref, g_ref, be_ref,
                   mavg_ref, f_ref, wl_ref, bl_ref, out_ref):
    hi = lax.Precision.HIGHEST
    t = (aggp_ref[0:NROW, :] + aggp_ref[PPAD:PPAD + NROW, :] + hs_ref[...]) \
        * dinv_ref[...] + _rep8(b_ref[...])[None, :]
    r = _ln_relu(t, mavg_ref, g_ref, be_ref)
    srow = jnp.sum(r, axis=0, keepdims=True)
    pooled = jnp.dot(srow, f_ref[...], precision=hi,
                     preferred_element_type=jnp.float32) * (1.0 / N)
    out_ref[...] = jnp.dot(pooled, wl_ref[...],
                           preferred_element_type=jnp.float32) \
        + bl_ref[...][None, :]


def _tc_final(aggp, hs, dinv, b, g, be, mavg, f, wl, bl):
    return pl.pallas_call(
        _tc_final_body,
        out_shape=jax.ShapeDtypeStruct((1, 25), jnp.float32),
    )(aggp, hs, dinv, b, g, be, mavg, f, wl, bl)


def kernel(x, edge_index, W1, b1, g1, be1, W2, b2, g2, be2, W3, b3, g3, be3,
           Wl, bl):
    e3 = edge_index.reshape(2, NW, G, C)
    x3 = x.reshape(NROW, 8, 128)
    eye8 = jnp.eye(8, dtype=jnp.float32)
    mavg = jnp.kron(eye8, jnp.full((16, 16), 1.0 / 16, jnp.float32))
    fmat = jnp.kron(jnp.ones((8, 1), jnp.float32),
                    jnp.eye(16, dtype=jnp.float32))

    degp = _deg_call(e3).reshape(NC * PPAD, 128)
    dinv, hs = _tc1(degp, x3, W1)

    aggp = _agg_call(hs.reshape(N, LANES), e3).reshape(NC * PPAD, 128)
    hs = _tc_mid(aggp, hs, dinv, b1, g1, be1, jnp.kron(eye8, W2), mavg)

    aggp = _agg_call(hs.reshape(N, LANES), e3).reshape(NC * PPAD, 128)
    hs = _tc_mid(aggp, hs, dinv, b2, g2, be2, jnp.kron(eye8, W3), mavg)

    aggp = _agg_call(hs.reshape(N, LANES), e3).reshape(NC * PPAD, 128)
    out = _tc_final(aggp, hs, dinv, b3, g3, be3, mavg, fmat, Wl, bl)
    return out.reshape(25)


# speedup vs baseline: 1.0651x; 1.0459x over previous
import functools

import jax
import jax.numpy as jnp
from jax import lax
from jax.experimental import pallas as pl
from jax.experimental.pallas import tpu as pltpu
from jax.experimental.pallas import tpu_sc as plsc

NC = 2
NS = 16
NW = NC * NS
LANES = 16

N = 10000
E = 320000
C = 80
EW = E // NW
G = EW // C
NPAD = 10240
RPS = NPAD // NS
NROW = N // 8
PPAD = NPAD // 8

NB = 12
PD = 7


def _sc_mesh():
    return plsc.VectorSubcoreMesh(
        core_axis_name="c", subcore_axis_name="s",
        num_cores=NC, num_subcores=NS)


def _deg_body(e_hbm, out_hbm, dst_slab, ones_v, stage, acc, sem):
    c = lax.axis_index("c")
    s = lax.axis_index("s")
    wid = c * NS + s
    dst_load = pltpu.async_copy(e_hbm.at[1].at[wid], dst_slab, sem)

    def zrow(i, _):
        stage[i] = jnp.zeros((LANES,), jnp.float32)
        return 0
    lax.fori_loop(0, RPS, zrow, 0)

    def orow(i, _):
        ones_v[i] = jnp.ones((LANES,), jnp.float32)
        return 0
    lax.fori_loop(0, C, orow, 0)

    pltpu.sync_copy(stage, acc.at[pl.ds(s * RPS, RPS)])
    dst_load.wait()
    plsc.subcore_barrier()

    def drain():
        pltpu.make_async_copy(out_hbm.at[pl.ds(0, C)], ones_v, sem).wait()

    def chunk_a(g, _):
        pltpu.async_copy(ones_v, acc.at[dst_slab.at[g]], sem, add=True)
        return 0

    def chunk_b(g, _):
        pltpu.async_copy(ones_v, acc.at[dst_slab.at[g]], sem, add=True)
        drain()
        return 0
    lax.fori_loop(0, NB, chunk_a, 0)
    lax.fori_loop(NB, G, chunk_b, 0)
    for _ in range(NB):
        drain()
    plsc.subcore_barrier()

    pltpu.sync_copy(acc.at[pl.ds(s * RPS, RPS)],
                    out_hbm.at[pl.ds(wid * RPS, RPS)])


@functools.lru_cache(maxsize=None)
def _deg_kernel():
    return pl.kernel(
        _deg_body,
        out_type=jax.ShapeDtypeStruct((NC * NPAD, LANES), jnp.float32),
        mesh=_sc_mesh(),
        scratch_types=[
            pltpu.VMEM((G, C), jnp.int32),
            pltpu.VMEM((C, LANES), jnp.float32),
            pltpu.VMEM((RPS, LANES), jnp.float32),
            pltpu.VMEM_SHARED((NPAD, LANES), jnp.float32),
            pltpu.SemaphoreType.DMA,
        ],
        compiler_params=pltpu.CompilerParams(use_tc_tiling_on_sc=False),
    )


def _deg_call(e3):
    return _deg_kernel()(e3)


def _agg_body(hs_hbm, e_hbm, out_hbm,
              src_slab, dst_slab, rows, stage, acc, gsem, ssem, zsem):
    c = lax.axis_index("c")
    s = lax.axis_index("s")
    wid = c * NS + s
    src_load = pltpu.async_copy(e_hbm.at[0].at[wid], src_slab, zsem)
    dst_load = pltpu.async_copy(e_hbm.at[1].at[wid], dst_slab, zsem)

    def zrow(i, _):
        stage[i] = jnp.zeros((LANES,), jnp.float32)
        return 0
    lax.fori_loop(0, RPS, zrow, 0)
    src_load.wait()
    dst_load.wait()

    def fire_gather(g, b):
        pltpu.async_copy(hs_hbm.at[src_slab.at[g]], rows.at[b], gsem.at[b])

    def drain(sem_arr, b):
        pltpu.make_async_copy(hs_hbm.at[pl.ds(0, C)], rows.at[b],
                              sem_arr.at[b]).wait()

    def scat(g, b):
        pltpu.async_copy(rows.at[b], acc.at[dst_slab.at[g]], ssem.at[b],
                         add=True)

    zero_copy = pltpu.async_copy(stage, acc.at[pl.ds(s * RPS, RPS)], zsem)
    for g in range(PD):
        fire_gather(jnp.int32(g), jnp.int32(g))
    zero_copy.wait()
    plsc.subcore_barrier()

    def body_a(g, _):
        b = lax.rem(g, NB)
        drain(gsem, b)
        scat(g, b)
        gp = g + PD
        fire_gather(gp, lax.rem(gp, NB))
        return 0

    def body_b(g, _):
        b = lax.rem(g, NB)
        drain(gsem, b)
        scat(g, b)
        gp = g + PD
        bp = lax.rem(gp, NB)
        drain(ssem, bp)
        fire_gather(gp, bp)
        return 0

    def body_c(g, _):
        b = lax.rem(g, NB)
        drain(gsem, b)
        scat(g, b)
        return 0
    lax.fori_loop(0, NB - PD, body_a, 0)
    lax.fori_loop(NB - PD, G - PD, body_b, 0)
    lax.fori_loop(G - PD, G, body_c, 0)
    for k in range(NB):
        drain(ssem, jnp.int32((G - NB + k) % NB))
    plsc.subcore_barrier()

    pltpu.sync_copy(acc.at[pl.ds(s * RPS, RPS)],
                    out_hbm.at[pl.ds(wid * RPS, RPS)])


@functools.lru_cache(maxsize=None)
def _agg_kernel():
    return pl.kernel(
        _agg_body,
        out_type=jax.ShapeDtypeStruct((NC * NPAD, LANES), jnp.float32),
        mesh=_sc_mesh(),
        scratch_types=[
            pltpu.VMEM((G, C), jnp.int32),
            pltpu.VMEM((G, C), jnp.int32),
            pltpu.VMEM((NB, C, LANES), jnp.float32),
            pltpu.VMEM((RPS, LANES), jnp.float32),
            pltpu.VMEM_SHARED((NPAD, LANES), jnp.float32),
            pltpu.SemaphoreType.DMA((NB,)),
            pltpu.SemaphoreType.DMA((NB,)),
            pltpu.SemaphoreType.DMA,
        ],
        compiler_params=pltpu.CompilerParams(use_tc_tiling_on_sc=False),
    )


def _agg_call(hs, e3):
    return _agg_kernel()(hs, e3)


def _tc1_body(degp_ref, x3_ref, w_ref, dinv_ref, hs_ref):
    deg = degp_ref[0:NROW, :] + degp_ref[PPAD:PPAD + NROW, :] + 1.0
    dinv = lax.rsqrt(deg)
    dinv_ref[...] = dinv
    for a in range(8):
        h = jnp.dot(x3_ref[:, a, :], w_ref[...],
                    preferred_element_type=jnp.float32)
        hs_ref[:, 16 * a:16 * (a + 1)] = h * dinv[:, 16 * a:16 * (a + 1)]


def _tc1(degp, x3, w):
    return pl.pallas_call(
        _tc1_body,
        out_shape=[
            jax.ShapeDtypeStruct((NROW, 128), jnp.float32),
            jax.ShapeDtypeStruct((NROW, 128), jnp.float32),
        ],
    )(degp, x3, w)


def _rep8(v):
    return jnp.concatenate([v] * 8, axis=0)


def _ln_relu(t, mavg_ref, g_ref, be_ref):
    hi = lax.Precision.HIGHEST
    mavg = mavg_ref[...]
    mu = jnp.dot(t, mavg, precision=hi, preferred_element_type=jnp.float32)
    d = t - mu
    var = jnp.dot(d * d, mavg, precision=hi,
                  preferred_element_type=jnp.float32)
    tn = d * lax.rsqrt(var + 1e-5) * _rep8(g_ref[...])[None, :] \
        + _rep8(be_ref[...])[None, :]
    return jnp.maximum(tn, 0.0)


def _tc_mid_body(aggp_ref, hs_ref, dinv_ref, b_ref, g_ref, be_ref,
                 wb_ref, mavg_ref, out_ref):
    dinv = dinv_ref[...]
    t = (aggp_ref[0:NROW, :] + aggp_ref[PPAD:PPAD + NROW, :] + hs_ref[...]) \
        * dinv + _rep8(b_ref[...])[None, :]
    r = _ln_relu(t, mavg_ref, g_ref, be_ref)
    out_ref[...] = jnp.dot(r, wb_ref[...],
                           preferred_element_type=jnp.float32) * dinv


def _tc_mid(aggp, hs, dinv, b, g, be, wb, mavg):
    return pl.pallas_call(
        _tc_mid_body,
        out_shape=jax.ShapeDtypeStruct((NROW, 128), jnp.float32),
    )(aggp, hs, dinv, b, g, be, wb, mavg)


def _tc_final_body(aggp_ref, hs_ref, dinv_ref, b_ref, g_ref, be_ref,
                   mavg_ref, f_ref, wl_ref, bl_ref, out_ref):
    hi = lax.Precision.HIGHEST
    t = (aggp_ref[0:NROW, :] + aggp_ref[PPAD:PPAD + NROW, :] + hs_ref[...]) \
        * dinv_ref[...] + _rep8(b_ref[...])[None, :]
    r = _ln_relu(t, mavg_ref, g_ref, be_ref)
    srow = jnp.sum(r, axis=0, keepdims=True)
    pooled = jnp.dot(srow, f_ref[...], precision=hi,
                     preferred_element_type=jnp.float32) * (1.0 / N)
    out_ref[...] = jnp.dot(pooled, wl_ref[...],
                           preferred_element_type=jnp.float32) \
        + bl_ref[...][None, :]


def _tc_final(aggp, hs, dinv, b, g, be, mavg, f, wl, bl):
    return pl.pallas_call(
        _tc_final_body,
        out_shape=jax.ShapeDtypeStruct((1, 25), jnp.float32),
    )(aggp, hs, dinv, b, g, be, mavg, f, wl, bl)


def kernel(x, edge_index, W1, b1, g1, be1, W2, b2, g2, be2, W3, b3, g3, be3,
           Wl, bl):
    e3 = edge_index.reshape(2, NW, G, C)
    x3 = x.reshape(NROW, 8, 128)
    eye8 = jnp.eye(8, dtype=jnp.float32)
    mavg = jnp.kron(eye8, jnp.full((16, 16), 1.0 / 16, jnp.float32))
    fmat = jnp.kron(jnp.ones((8, 1), jnp.float32),
                    jnp.eye(16, dtype=jnp.float32))

    degp = _deg_call(e3).reshape(NC * PPAD, 128)
    dinv, hs = _tc1(degp, x3, W1)

    aggp = _agg_call(hs.reshape(N, LANES), e3).reshape(NC * PPAD, 128)
    hs = _tc_mid(aggp, hs, dinv, b1, g1, be1, jnp.kron(eye8, W2), mavg)

    aggp = _agg_call(hs.reshape(N, LANES), e3).reshape(NC * PPAD, 128)
    hs = _tc_mid(aggp, hs, dinv, b2, g2, be2, jnp.kron(eye8, W3), mavg)

    aggp = _agg_call(hs.reshape(N, LANES), e3).reshape(NC * PPAD, 128)
    out = _tc_final(aggp, hs, dinv, b3, g3, be3, mavg, fmat, Wl, bl)
    return out.reshape(25)


# speedup vs baseline: 1.0981x; 1.0309x over previous
import functools

import jax
import jax.numpy as jnp
from jax import lax
from jax.experimental import pallas as pl
from jax.experimental.pallas import tpu as pltpu
from jax.experimental.pallas import tpu_sc as plsc

NC = 2
NS = 16
NW = NC * NS
LANES = 16

N = 10000
E = 320000
C = 80
EW = E // NW
G = EW // C
NPAD = 10240
RPS = NPAD // NS
NROW = N // 8
PPAD = NPAD // 8

NB = 12
PD = 8


def _sc_mesh():
    return plsc.VectorSubcoreMesh(
        core_axis_name="c", subcore_axis_name="s",
        num_cores=NC, num_subcores=NS)


def _deg_body(e_hbm, out_hbm, dst_slab, ones_v, stage, acc, sem):
    c = lax.axis_index("c")
    s = lax.axis_index("s")
    wid = c * NS + s
    dst_load = pltpu.async_copy(e_hbm.at[1].at[wid], dst_slab, sem)

    def zrow(i, _):
        stage[i] = jnp.zeros((LANES,), jnp.float32)
        return 0
    lax.fori_loop(0, RPS, zrow, 0)

    def orow(i, _):
        ones_v[i] = jnp.ones((LANES,), jnp.float32)
        return 0
    lax.fori_loop(0, C, orow, 0)

    pltpu.sync_copy(stage, acc.at[pl.ds(s * RPS, RPS)])
    dst_load.wait()
    plsc.subcore_barrier()

    def drain():
        pltpu.make_async_copy(out_hbm.at[pl.ds(0, C)], ones_v, sem).wait()

    def chunk_a(g, _):
        pltpu.async_copy(ones_v, acc.at[dst_slab.at[g]], sem, add=True)
        return 0

    def chunk_b(g, _):
        pltpu.async_copy(ones_v, acc.at[dst_slab.at[g]], sem, add=True)
        drain()
        return 0
    lax.fori_loop(0, NB, chunk_a, 0)
    lax.fori_loop(NB, G, chunk_b, 0)
    for _ in range(NB):
        drain()
    plsc.subcore_barrier()

    pltpu.sync_copy(acc.at[pl.ds(s * RPS, RPS)],
                    out_hbm.at[pl.ds(wid * RPS, RPS)])


@functools.lru_cache(maxsize=None)
def _deg_kernel():
    return pl.kernel(
        _deg_body,
        out_type=jax.ShapeDtypeStruct((NC * NPAD, LANES), jnp.float32),
        mesh=_sc_mesh(),
        scratch_types=[
            pltpu.VMEM((G, C), jnp.int32),
            pltpu.VMEM((C, LANES), jnp.float32),
            pltpu.VMEM((RPS, LANES), jnp.float32),
            pltpu.VMEM_SHARED((NPAD, LANES), jnp.float32),
            pltpu.SemaphoreType.DMA,
        ],
        compiler_params=pltpu.CompilerParams(use_tc_tiling_on_sc=False),
    )


def _deg_call(e3):
    return _deg_kernel()(e3)


def _agg_body(hs_hbm, e_hbm, out_hbm,
              src_slab, dst_slab, rows, stage, acc, gsem, ssem, zsem):
    c = lax.axis_index("c")
    s = lax.axis_index("s")
    wid = c * NS + s
    src_load = pltpu.async_copy(e_hbm.at[0].at[wid], src_slab, zsem)
    dst_load = pltpu.async_copy(e_hbm.at[1].at[wid], dst_slab, zsem)

    def zrow(i, _):
        stage[i] = jnp.zeros((LANES,), jnp.float32)
        return 0
    lax.fori_loop(0, RPS, zrow, 0)
    src_load.wait()
    dst_load.wait()

    def fire_gather(g, b):
        pltpu.async_copy(hs_hbm.at[src_slab.at[g]], rows.at[b], gsem.at[b])

    def drain(sem_arr, b):
        pltpu.make_async_copy(hs_hbm.at[pl.ds(0, C)], rows.at[b],
                              sem_arr.at[b]).wait()

    def scat(g, b):
        pltpu.async_copy(rows.at[b], acc.at[dst_slab.at[g]], ssem.at[b],
                         add=True)

    zero_copy = pltpu.async_copy(stage, acc.at[pl.ds(s * RPS, RPS)], zsem)
    for g in range(PD):
        fire_gather(jnp.int32(g), jnp.int32(g))
    zero_copy.wait()
    plsc.subcore_barrier()

    def body_a(g, _):
        b = lax.rem(g, NB)
        drain(gsem, b)
        scat(g, b)
        gp = g + PD
        fire_gather(gp, lax.rem(gp, NB))
        return 0

    def body_b(g, _):
        b = lax.rem(g, NB)
        drain(gsem, b)
        scat(g, b)
        gp = g + PD
        bp = lax.rem(gp, NB)
        drain(ssem, bp)
        fire_gather(gp, bp)
        return 0

    def body_c(g, _):
        b = lax.rem(g, NB)
        drain(gsem, b)
        scat(g, b)
        return 0
    lax.fori_loop(0, NB - PD, body_a, 0)
    lax.fori_loop(NB - PD, G - PD, body_b, 0)
    lax.fori_loop(G - PD, G, body_c, 0)
    for k in range(NB):
        drain(ssem, jnp.int32((G - NB + k) % NB))
    plsc.subcore_barrier()

    pltpu.sync_copy(acc.at[pl.ds(s * RPS, RPS)],
                    out_hbm.at[pl.ds(wid * RPS, RPS)])


@functools.lru_cache(maxsize=None)
def _agg_kernel():
    return pl.kernel(
        _agg_body,
        out_type=jax.ShapeDtypeStruct((NC * NPAD, LANES), jnp.float32),
        mesh=_sc_mesh(),
        scratch_types=[
            pltpu.VMEM((G, C), jnp.int32),
            pltpu.VMEM((G, C), jnp.int32),
            pltpu.VMEM((NB, C, LANES), jnp.float32),
            pltpu.VMEM((RPS, LANES), jnp.float32),
            pltpu.VMEM_SHARED((NPAD, LANES), jnp.float32),
            pltpu.SemaphoreType.DMA((NB,)),
            pltpu.SemaphoreType.DMA((NB,)),
            pltpu.SemaphoreType.DMA,
        ],
        compiler_params=pltpu.CompilerParams(use_tc_tiling_on_sc=False),
    )


def _agg_call(hs, e3):
    return _agg_kernel()(hs, e3)


def _tc1_body(degp_ref, x3_ref, w_ref, dinv_ref, hs_ref):
    deg = degp_ref[0:NROW, :] + degp_ref[PPAD:PPAD + NROW, :] + 1.0
    dinv = lax.rsqrt(deg)
    dinv_ref[...] = dinv
    for a in range(8):
        h = jnp.dot(x3_ref[:, a, :], w_ref[...],
                    preferred_element_type=jnp.float32)
        hs_ref[:, 16 * a:16 * (a + 1)] = h * dinv[:, 16 * a:16 * (a + 1)]


def _tc1(degp, x3, w):
    return pl.pallas_call(
        _tc1_body,
        out_shape=[
            jax.ShapeDtypeStruct((NROW, 128), jnp.float32),
            jax.ShapeDtypeStruct((NROW, 128), jnp.float32),
        ],
    )(degp, x3, w)


def _rep8(v):
    return jnp.concatenate([v] * 8, axis=0)


def _ln_relu(t, mavg_ref, g_ref, be_ref):
    hi = lax.Precision.HIGHEST
    mavg = mavg_ref[...]
    mu = jnp.dot(t, mavg, precision=hi, preferred_element_type=jnp.float32)
    d = t - mu
    var = jnp.dot(d * d, mavg, precision=hi,
                  preferred_element_type=jnp.float32)
    tn = d * lax.rsqrt(var + 1e-5) * _rep8(g_ref[...])[None, :] \
        + _rep8(be_ref[...])[None, :]
    return jnp.maximum(tn, 0.0)


def _tc_mid_body(aggp_ref, hs_ref, dinv_ref, b_ref, g_ref, be_ref,
                 wb_ref, mavg_ref, out_ref):
    dinv = dinv_ref[...]
    t = (aggp_ref[0:NROW, :] + aggp_ref[PPAD:PPAD + NROW, :] + hs_ref[...]) \
        * dinv + _rep8(b_ref[...])[None, :]
    r = _ln_relu(t, mavg_ref, g_ref, be_ref)
    out_ref[...] = jnp.dot(r, wb_ref[...],
                           preferred_element_type=jnp.float32) * dinv


def _tc_mid(aggp, hs, dinv, b, g, be, wb, mavg):
    return pl.pallas_call(
        _tc_mid_body,
        out_shape=jax.ShapeDtypeStruct((NROW, 128), jnp.float32),
    )(aggp, hs, dinv, b, g, be, wb, mavg)


def _tc_final_body(aggp_ref, hs_ref, dinv_ref, b_ref, g_ref, be_ref,
                   mavg_ref, f_ref, wl_ref, bl_ref, out_ref):
    hi = lax.Precision.HIGHEST
    t = (aggp_ref[0:NROW, :] + aggp_ref[PPAD:PPAD + NROW, :] + hs_ref[...]) \
        * dinv_ref[...] + _rep8(b_ref[...])[None, :]
    r = _ln_relu(t, mavg_ref, g_ref, be_ref)
    srow = jnp.sum(r, axis=0, keepdims=True)
    pooled = jnp.dot(srow, f_ref[...], precision=hi,
                     preferred_element_type=jnp.float32) * (1.0 / N)
    out_ref[...] = jnp.dot(pooled, wl_ref[...],
                           preferred_element_type=jnp.float32) \
        + bl_ref[...][None, :]


def _tc_final(aggp, hs, dinv, b, g, be, mavg, f, wl, bl):
    return pl.pallas_call(
        _tc_final_body,
        out_shape=jax.ShapeDtypeStruct((1, 25), jnp.float32),
    )(aggp, hs, dinv, b, g, be, mavg, f, wl, bl)


def kernel(x, edge_index, W1, b1, g1, be1, W2, b2, g2, be2, W3, b3, g3, be3,
           Wl, bl):
    e3 = edge_index.reshape(2, NW, G, C)
    x3 = x.reshape(NROW, 8, 128)
    eye8 = jnp.eye(8, dtype=jnp.float32)
    mavg = jnp.kron(eye8, jnp.full((16, 16), 1.0 / 16, jnp.float32))
    fmat = jnp.kron(jnp.ones((8, 1), jnp.float32),
                    jnp.eye(16, dtype=jnp.float32))

    degp = _deg_call(e3).reshape(NC * PPAD, 128)
    dinv, hs = _tc1(degp, x3, W1)

    aggp = _agg_call(hs.reshape(N, LANES), e3).reshape(NC * PPAD, 128)
    hs = _tc_mid(aggp, hs, dinv, b1, g1, be1, jnp.kron(eye8, W2), mavg)

    aggp = _agg_call(hs.reshape(N, LANES), e3).reshape(NC * PPAD, 128)
    hs = _tc_mid(aggp, hs, dinv, b2, g2, be2, jnp.kron(eye8, W3), mavg)

    aggp = _agg_call(hs.reshape(N, LANES), e3).reshape(NC * PPAD, 128)
    out = _tc_final(aggp, hs, dinv, b3, g3, be3, mavg, fmat, Wl, bl)
    return out.reshape(25)
